# baseline (speedup 1.0000x reference)
import functools

import numpy as np

import jax
import jax.numpy as jnp
from jax import lax
from jax.experimental import pallas as pl
from jax.experimental.pallas import tpu as pltpu

NEG_SLOPE = 0.2
BN_EPS = 1e-5
FEAT = 120
LANES = 128
CPP = 24

W2_OFF = 48
WFC_OFF = 336
B1_ROW = 368
B2_ROW = 369
BFC_ROW = 370
GAMMA_ROW = 371
BETA_ROW = 372

W1T_OFF = 0
W2T_OFF = 600
WFCT_OFF = 696
WTS_ROWS = 832


def _leaky(v):
    return jnp.where(v > 0, v, NEG_SLOPE * v)


def _tap(p, l):
    h, w = divmod(p, 5)
    ci, r = divmod(l, 25)
    ih, iw = divmod(r, 5)
    kh, kw = ih - h + 1, iw - w + 1
    if 0 <= kh < 3 and 0 <= kw < 3:
        return (kh * 3 + kw) * 5 + ci
    return 45


def _w1_row_index():
    idx = np.full((25, 128), 45, np.int32)
    for p in range(25):
        for l in range(125):
            idx[p, l] = _tap(p, l)
    return idx.reshape(25 * 128)


_W1_IDX = _w1_row_index()


def _pass1(state_ref, w_ref, w2b_ref, y_ref, stats_ref, pool_ref, *, tb, n_valid):
    x = state_ref[...]
    x = jnp.pad(x, ((0, 0), (0, LANES - x.shape[1])))
    lane = lax.broadcasted_iota(jnp.int32, x.shape, 1)
    x = jnp.where(lane == 125, 1.0, x)
    xt = jnp.transpose(x)

    c1 = jnp.dot(w_ref[W1T_OFF:W1T_OFF + 25 * CPP, :], xt,
                 preferred_element_type=jnp.float32)

    hm = {}
    for ph in range(5):
        for pw in range(4):
            a0 = (ph * 5 + pw) * CPP
            hm[(ph, pw)] = jnp.maximum(c1[a0:a0 + CPP], c1[a0 + CPP:a0 + 2 * CPP])
    for ph in range(4):
        for pw in range(4):
            pool_ref[(ph * 4 + pw) * CPP:(ph * 4 + pw + 1) * CPP, :] = \
                _leaky(jnp.maximum(hm[(ph, pw)], hm[(ph + 1, pw)]))

    c2 = jnp.dot(w2b_ref[...], pool_ref[...],
                 preferred_element_type=jnp.float32)

    feat = jnp.maximum(jnp.maximum(c2[0:32], c2[32:64]),
                       jnp.maximum(c2[64:96], c2[96:128]))
    feat = _leaky(feat + w_ref[W2T_OFF:W2T_OFF + 32, 96:97])

    rid = lax.broadcasted_iota(jnp.int32, feat.shape, 0)
    feat = jnp.where(rid == 30, 1.0, feat)
    y = jnp.dot(w_ref[WFCT_OFF:WFCT_OFF + 128, 0:32], feat,
                preferred_element_type=jnp.float32)
    y_ref[...] = y.astype(jnp.bfloat16)

    if n_valid is None:
        yv = y
    else:
        gid = pl.program_id(0) * tb + lax.broadcasted_iota(jnp.int32, y.shape, 1)
        yv = jnp.where(gid < n_valid, y, 0.0)
    s = jnp.sum(yv, axis=1, keepdims=True)
    ss = jnp.sum(yv * y, axis=1, keepdims=True)
    lid = lax.broadcasted_iota(jnp.int32, stats_ref.shape, 1)
    stats_ref[...] = jnp.where(lid == 0, s, 0.0) + jnp.where(lid == 1, ss, 0.0)


def _pass2(y_ref, aux_ref, out_ref, *, inv_n):
    mean = aux_ref[:, 0:1] * inv_n
    var = jnp.maximum(aux_ref[:, 1:2] * inv_n - mean * mean, 0.0)
    y_hat = (y_ref[...].astype(jnp.float32) - mean) * lax.rsqrt(var + BN_EPS)
    o = _leaky(y_hat * aux_ref[:, 2:3] + aux_ref[:, 3:4])
    out_ref[...] = jnp.transpose(o)[:, 0:FEAT]


_MOSAIC = dict(vmem_limit_bytes=48 * 1024 * 1024)


@jax.jit
def _forward(state, slab):
    B = state.shape[0]
    b_pad = ((B + 7) // 8) * 8
    tb = next(t for t in (2048, 1024, 512, 256, 128) if b_pad % t == 0) \
        if b_pad % 128 == 0 else b_pad
    nb = b_pad // tb
    if b_pad != B:
        state = jnp.pad(state, ((0, b_pad - B), (0, 0)))
    n_valid = None if b_pad == B else B
    lane = jnp.arange(LANES)[None, :]

    w1u = jnp.take(slab[0:48, 0:32], _W1_IDX, axis=0)
    w1t = w1u.reshape(25, 128, 32).transpose(0, 2, 1)[:, 0:CPP, :]
    w1t = w1t.reshape(25 * CPP, 128)
    w1t = jnp.where(lane == 125,
                    jnp.tile(slab[B1_ROW, 0:CPP], 25)[:, None], w1t)
    w2r = slab[W2_OFF:WFC_OFF, 0:32].reshape(3, 3, 32, 32)
    w2r = w2r.transpose(0, 3, 1, 2)[:, :, :, 0:CPP].reshape(3, 32, 3 * CPP)
    w2t = jnp.pad(w2r.reshape(96, 3 * CPP), ((0, 0), (0, LANES - 3 * CPP)))
    w2t = jnp.where(lane == 96, jnp.tile(slab[B2_ROW, 0:32], 3)[:, None], w2t)
    blocks = []
    for oh in range(2):
        for ow in range(2):
            acc = jnp.zeros((32, 16 * CPP), jnp.float32)
            for kh in range(3):
                off = (oh + kh) * 4 * CPP + ow * CPP
                acc = acc + jnp.pad(
                    w2r[kh], ((0, 0), (off, 16 * CPP - off - 3 * CPP)))
            blocks.append(acc)
    w2big = jnp.concatenate(blocks, axis=0)
    wfct = jnp.transpose(slab[WFC_OFF:WFC_OFF + 32, :])
    wfct = jnp.where(jnp.arange(32)[None, :] == 30, slab[BFC_ROW, :][:, None],
                     wfct)
    wfct = jnp.pad(wfct, ((0, 0), (0, 96)))
    wts = jnp.concatenate(
        [w1t, w2t, wfct,
         jnp.zeros((WTS_ROWS - 824, LANES), jnp.float32)], axis=0)

    flops = 2 * b_pad * (600 * LANES + 12 * 32 * 72 + LANES * 32)
    bytes1 = 4 * (b_pad * 125 + WTS_ROWS * LANES + nb * LANES * LANES) \
        + 2 * b_pad * LANES
    y, stats = pl.pallas_call(
        functools.partial(_pass1, tb=tb, n_valid=n_valid),
        grid=(nb,),
        in_specs=[pl.BlockSpec((tb, 125), lambda i: (i, 0)),
                  pl.BlockSpec((WTS_ROWS, LANES), lambda i: (0, 0)),
                  pl.BlockSpec((LANES, 16 * CPP), lambda i: (0, 0))],
        out_specs=[pl.BlockSpec((LANES, tb), lambda i: (0, i)),
                   pl.BlockSpec((LANES, LANES), lambda i: (0, i))],
        out_shape=(jax.ShapeDtypeStruct((LANES, b_pad), jnp.bfloat16),
                   jax.ShapeDtypeStruct((LANES, nb * LANES), jnp.float32)),
        scratch_shapes=[pltpu.VMEM((16 * CPP, tb), jnp.float32)],
        compiler_params=pltpu.CompilerParams(
            dimension_semantics=("parallel",), **_MOSAIC),
        cost_estimate=pl.CostEstimate(flops=flops, transcendentals=0,
                                      bytes_accessed=bytes1),
    )(state, wts, w2big)

    totals = jnp.sum(stats.reshape(LANES, nb, LANES), axis=1)
    aux = jnp.concatenate(
        [totals[:, 0:1], totals[:, 1:2],
         slab[GAMMA_ROW, :][:, None], slab[BETA_ROW, :][:, None]], axis=1)
    aux = jnp.pad(aux, ((0, 0), (0, LANES - 4)))

    tb2 = tb * 8 if b_pad % (tb * 8) == 0 else tb
    nb2 = b_pad // tb2
    out = pl.pallas_call(
        functools.partial(_pass2, inv_n=1.0 / B),
        grid=(nb2,),
        in_specs=[pl.BlockSpec((LANES, tb2), lambda i: (0, i)),
                  pl.BlockSpec((LANES, LANES), lambda i: (0, 0))],
        out_specs=pl.BlockSpec((tb2, FEAT), lambda i: (i, 0)),
        out_shape=jax.ShapeDtypeStruct((b_pad, FEAT), jnp.float32),
        compiler_params=pltpu.CompilerParams(
            dimension_semantics=("parallel",), **_MOSAIC),
        cost_estimate=pl.CostEstimate(
            flops=10 * b_pad * LANES, transcendentals=LANES,
            bytes_accessed=4 * b_pad * FEAT + 2 * b_pad * LANES),
    )(y, aux)

    return out[:B] if b_pad != B else out


def kernel(state, slab):
    return _forward(state, slab)


# speedup vs baseline: 1.0481x; 1.0481x over previous
import functools

import numpy as np

import jax
import jax.numpy as jnp
from jax import lax
from jax.experimental import pallas as pl
from jax.experimental.pallas import tpu as pltpu

NEG_SLOPE = 0.2
BN_EPS = 1e-5
FEAT = 120
LANES = 128
CPP = 24

W2_OFF = 48
WFC_OFF = 336
B1_ROW = 368
B2_ROW = 369
BFC_ROW = 370
GAMMA_ROW = 371
BETA_ROW = 372

W1T_OFF = 0
W2T_OFF = 600
WFCT_OFF = 696
WTS_ROWS = 832


def _leaky(v):
    return jnp.where(v > 0, v, NEG_SLOPE * v)


def _tap(p, l):
    h, w = divmod(p, 5)
    ci, r = divmod(l, 25)
    ih, iw = divmod(r, 5)
    kh, kw = ih - h + 1, iw - w + 1
    if 0 <= kh < 3 and 0 <= kw < 3:
        return (kh * 3 + kw) * 5 + ci
    return 45


def _w1_row_index():
    idx = np.full((25, 128), 45, np.int32)
    for p in range(25):
        for l in range(125):
            idx[p, l] = _tap(p, l)
    return idx.reshape(25 * 128)


_W1_IDX = _w1_row_index()


def _pass1(state_ref, w_ref, w2b_ref, y_ref, stats_ref, pool_ref, *, tb, n_valid):
    x = state_ref[...]
    x = jnp.pad(x, ((0, 0), (0, LANES - x.shape[1])))
    lane = lax.broadcasted_iota(jnp.int32, x.shape, 1)
    x = jnp.where(lane == 125, 1.0, x)
    xt = jnp.transpose(x)

    c1 = jnp.dot(w_ref[W1T_OFF:W1T_OFF + 25 * CPP, :], xt,
                 preferred_element_type=jnp.float32)

    hm = {}
    for ph in range(5):
        for pw in range(4):
            a0 = (ph * 5 + pw) * CPP
            hm[(ph, pw)] = jnp.maximum(c1[a0:a0 + CPP], c1[a0 + CPP:a0 + 2 * CPP])
    for ph in range(4):
        for pw in range(4):
            pool_ref[(ph * 4 + pw) * CPP:(ph * 4 + pw + 1) * CPP, :] = \
                _leaky(jnp.maximum(hm[(ph, pw)], hm[(ph + 1, pw)]))

    c2 = jnp.dot(w2b_ref[...], pool_ref[...],
                 preferred_element_type=jnp.float32)

    feat = jnp.maximum(jnp.maximum(c2[0:32], c2[32:64]),
                       jnp.maximum(c2[64:96], c2[96:128]))
    feat = _leaky(feat + w_ref[W2T_OFF:W2T_OFF + 32, 96:97])

    rid = lax.broadcasted_iota(jnp.int32, feat.shape, 0)
    feat = jnp.where(rid == 30, 1.0, feat)
    y = jnp.dot(w_ref[WFCT_OFF:WFCT_OFF + 128, 0:32], feat,
                preferred_element_type=jnp.float32)
    y_ref[...] = y.astype(jnp.bfloat16)

    if n_valid is None:
        yv = y
    else:
        gid = pl.program_id(0) * tb + lax.broadcasted_iota(jnp.int32, y.shape, 1)
        yv = jnp.where(gid < n_valid, y, 0.0)
    s = jnp.sum(yv, axis=1, keepdims=True)
    ss = jnp.sum(yv * y, axis=1, keepdims=True)
    lid = lax.broadcasted_iota(jnp.int32, stats_ref.shape, 1)
    stats_ref[...] = jnp.where(lid == 0, s, 0.0) + jnp.where(lid == 1, ss, 0.0)


def _pass2(y_ref, aux_ref, out_ref, *, inv_n):
    mean = aux_ref[:, 0:1] * inv_n
    var = jnp.maximum(aux_ref[:, 1:2] * inv_n - mean * mean, 0.0)
    y_hat = (y_ref[...].astype(jnp.float32) - mean) * lax.rsqrt(var + BN_EPS)
    o = _leaky(y_hat * aux_ref[:, 2:3] + aux_ref[:, 3:4])
    out_ref[...] = jnp.transpose(o)[:, 0:FEAT]


_MOSAIC = dict(vmem_limit_bytes=48 * 1024 * 1024)


@jax.jit
def _forward(state, slab):
    B = state.shape[0]
    b_pad = ((B + 7) // 8) * 8
    tb = next(t for t in (4096, 2048, 1024, 512, 256, 128) if b_pad % t == 0) \
        if b_pad % 128 == 0 else b_pad
    nb = b_pad // tb
    if b_pad != B:
        state = jnp.pad(state, ((0, b_pad - B), (0, 0)))
    n_valid = None if b_pad == B else B
    lane = jnp.arange(LANES)[None, :]

    w1u = jnp.take(slab[0:48, 0:32], _W1_IDX, axis=0)
    w1t = w1u.reshape(25, 128, 32).transpose(0, 2, 1)[:, 0:CPP, :]
    w1t = w1t.reshape(25 * CPP, 128)
    w1t = jnp.where(lane == 125,
                    jnp.tile(slab[B1_ROW, 0:CPP], 25)[:, None], w1t)
    w2r = slab[W2_OFF:WFC_OFF, 0:32].reshape(3, 3, 32, 32)
    w2r = w2r.transpose(0, 3, 1, 2)[:, :, :, 0:CPP].reshape(3, 32, 3 * CPP)
    w2t = jnp.pad(w2r.reshape(96, 3 * CPP), ((0, 0), (0, LANES - 3 * CPP)))
    w2t = jnp.where(lane == 96, jnp.tile(slab[B2_ROW, 0:32], 3)[:, None], w2t)
    blocks = []
    for oh in range(2):
        for ow in range(2):
            acc = jnp.zeros((32, 16 * CPP), jnp.float32)
            for kh in range(3):
                off = (oh + kh) * 4 * CPP + ow * CPP
                acc = acc + jnp.pad(
                    w2r[kh], ((0, 0), (off, 16 * CPP - off - 3 * CPP)))
            blocks.append(acc)
    w2big = jnp.concatenate(blocks, axis=0)
    wfct = jnp.transpose(slab[WFC_OFF:WFC_OFF + 32, :])
    wfct = jnp.where(jnp.arange(32)[None, :] == 30, slab[BFC_ROW, :][:, None],
                     wfct)
    wfct = jnp.pad(wfct, ((0, 0), (0, 96)))
    wts = jnp.concatenate(
        [w1t, w2t, wfct,
         jnp.zeros((WTS_ROWS - 824, LANES), jnp.float32)], axis=0)

    flops = 2 * b_pad * (600 * LANES + 12 * 32 * 72 + LANES * 32)
    bytes1 = 4 * (b_pad * 125 + WTS_ROWS * LANES + nb * LANES * LANES) \
        + 2 * b_pad * LANES
    y, stats = pl.pallas_call(
        functools.partial(_pass1, tb=tb, n_valid=n_valid),
        grid=(nb,),
        in_specs=[pl.BlockSpec((tb, 125), lambda i: (i, 0)),
                  pl.BlockSpec((WTS_ROWS, LANES), lambda i: (0, 0)),
                  pl.BlockSpec((LANES, 16 * CPP), lambda i: (0, 0))],
        out_specs=[pl.BlockSpec((LANES, tb), lambda i: (0, i)),
                   pl.BlockSpec((LANES, LANES), lambda i: (0, i))],
        out_shape=(jax.ShapeDtypeStruct((LANES, b_pad), jnp.bfloat16),
                   jax.ShapeDtypeStruct((LANES, nb * LANES), jnp.float32)),
        scratch_shapes=[pltpu.VMEM((16 * CPP, tb), jnp.float32)],
        compiler_params=pltpu.CompilerParams(
            dimension_semantics=("parallel",), **_MOSAIC),
        cost_estimate=pl.CostEstimate(flops=flops, transcendentals=0,
                                      bytes_accessed=bytes1),
    )(state, wts, w2big)

    totals = jnp.sum(stats.reshape(LANES, nb, LANES), axis=1)
    aux = jnp.concatenate(
        [totals[:, 0:1], totals[:, 1:2],
         slab[GAMMA_ROW, :][:, None], slab[BETA_ROW, :][:, None]], axis=1)
    aux = jnp.pad(aux, ((0, 0), (0, LANES - 4)))

    tb2 = tb * 4 if b_pad % (tb * 4) == 0 else tb
    nb2 = b_pad // tb2
    out = pl.pallas_call(
        functools.partial(_pass2, inv_n=1.0 / B),
        grid=(nb2,),
        in_specs=[pl.BlockSpec((LANES, tb2), lambda i: (0, i)),
                  pl.BlockSpec((LANES, LANES), lambda i: (0, 0))],
        out_specs=pl.BlockSpec((tb2, FEAT), lambda i: (i, 0)),
        out_shape=jax.ShapeDtypeStruct((b_pad, FEAT), jnp.float32),
        compiler_params=pltpu.CompilerParams(
            dimension_semantics=("parallel",), **_MOSAIC),
        cost_estimate=pl.CostEstimate(
            flops=10 * b_pad * LANES, transcendentals=LANES,
            bytes_accessed=4 * b_pad * FEAT + 2 * b_pad * LANES),
    )(y, aux)

    return out[:B] if b_pad != B else out


def kernel(state, slab):
    return _forward(state, slab)
